# baseline (speedup 1.0000x reference)
import functools

import jax
import jax.numpy as jnp
from jax.experimental import pallas as pl
from jax.experimental.pallas import tpu as pltpu

T = 2048
D = 768
E = 8
K = 2
TB = 256


def _router_kernel(x_ref, wr_ref, widx_ref, w_ref, comb_ref, rank_ref,
                   cnt_ref, aux_ref):
  x = x_ref[...]
  logits = jnp.dot(x, wr_ref[...], preferred_element_type=jnp.float32)
  m = jnp.max(logits, axis=-1, keepdims=True)
  p = jnp.exp(logits - m)
  p = p / jnp.sum(p, axis=-1, keepdims=True)

  lane = jax.lax.broadcasted_iota(jnp.int32, (T, E), 1)
  e0 = jnp.argmax(p, axis=-1).astype(jnp.int32)
  p0 = jnp.max(p, axis=-1)
  oh0 = (lane == e0[:, None]).astype(jnp.float32)
  p_m = jnp.where(oh0 > 0, -jnp.inf, p)
  e1 = jnp.argmax(p_m, axis=-1).astype(jnp.int32)
  p1 = jnp.max(p_m, axis=-1)
  oh1 = (lane == e1[:, None]).astype(jnp.float32)

  s = p0 + p1
  w0 = p0 / s
  w1 = p1 / s

  comb = oh0 * w0[:, None] + oh1 * w1[:, None]
  comb_ref[...] = comb

  widx_ref[...] = jnp.concatenate([e0[:, None], e1[:, None]], axis=1)
  w_ref[...] = jnp.concatenate([w0[:, None], w1[:, None]], axis=1)

  cnt2 = oh0 + oh1
  r = jax.lax.broadcasted_iota(jnp.int32, (TB, TB), 0)
  c = jax.lax.broadcasted_iota(jnp.int32, (TB, TB), 1)
  tri = (r > c).astype(jnp.float32)
  carry = jnp.zeros((1, E), jnp.float32)
  ce_blocks = []
  for i in range(T // TB):
    blk = cnt2[i * TB:(i + 1) * TB]
    ce_blocks.append(jnp.dot(tri, blk, preferred_element_type=jnp.float32)
                     + carry)
    carry = carry + jnp.sum(blk, axis=0, keepdims=True)
  ce = jnp.concatenate(ce_blocks, axis=0)

  rank0 = jnp.sum(ce * oh0, axis=-1)
  rank1 = jnp.sum(ce * oh1, axis=-1)
  rank_ref[...] = jnp.concatenate(
      [rank0[:, None], rank1[:, None]], axis=1).astype(jnp.int32)
  cnt_ref[...] = carry

  density = carry / T
  importance = jnp.sum(comb, axis=0, keepdims=True) / T
  aux_ref[...] = E * jnp.sum(density * importance, axis=-1, keepdims=True)


def _router(x, wr):
  return pl.pallas_call(
      _router_kernel,
      out_shape=(
          jax.ShapeDtypeStruct((T, K), jnp.int32),
          jax.ShapeDtypeStruct((T, K), jnp.float32),
          jax.ShapeDtypeStruct((T, E), jnp.float32),
          jax.ShapeDtypeStruct((T, K), jnp.int32),
          jax.ShapeDtypeStruct((1, E), jnp.float32),
          jax.ShapeDtypeStruct((1, 1), jnp.float32),
      ),
  )(x, wr)


def _dense_kernel(x_ref, w1_ref, b1_ref, w2_ref, b2_ref, scale_ref, out_ref):
  i = pl.program_id(0)
  x = x_ref[...]
  h = jax.nn.gelu(
      jnp.dot(x, w1_ref[0], preferred_element_type=jnp.float32) + b1_ref[0])
  y = jnp.dot(h, w2_ref[0], preferred_element_type=jnp.float32) + b2_ref[0]
  n = scale_ref.shape[1]
  lane = jax.lax.broadcasted_iota(jnp.int32, (T, n), 1)
  sc = jnp.sum(jnp.where(lane == i, scale_ref[...], 0.0), axis=1,
               keepdims=True)

  @pl.when(i == 0)
  def _():
    out_ref[...] = jnp.zeros_like(out_ref)

  out_ref[...] += y * sc


def _dense(x, w1all, b1all, w2all, b2all, scale):
  n = w1all.shape[0]
  return pl.pallas_call(
      _dense_kernel,
      grid=(n,),
      in_specs=[
          pl.BlockSpec((T, D), lambda i: (0, 0)),
          pl.BlockSpec((1, D, D), lambda i: (i, 0, 0)),
          pl.BlockSpec((1, 1, D), lambda i: (i, 0, 0)),
          pl.BlockSpec((1, D, D), lambda i: (i, 0, 0)),
          pl.BlockSpec((1, 1, D), lambda i: (i, 0, 0)),
          pl.BlockSpec((T, n), lambda i: (0, 0)),
      ],
      out_specs=pl.BlockSpec((T, D), lambda i: (0, 0)),
      out_shape=jax.ShapeDtypeStruct((T, D), jnp.float32),
  )(x, w1all, b1all[:, None, :], w2all, b2all[:, None, :], scale)


@jax.jit
def kernel(X, Wr, Ws1, bs1, Ws2, bs2, We1, be1, We2, be2):
  x = X[0]
  widx, w, comb, rank, cnt, aux = _router(x, Wr)

  w1all = jnp.concatenate([Ws1, We1], axis=0)
  b1all = jnp.concatenate([bs1, be1], axis=0)
  w2all = jnp.concatenate([Ws2, We2], axis=0)
  b2all = jnp.concatenate([bs2, be2], axis=0)
  scale = jnp.concatenate([jnp.ones((T, 1), jnp.float32), comb], axis=1)

  out = _dense(x, w1all, b1all, w2all, b2all, scale)
  return out[None], aux[0, 0]
